# baseline (speedup 1.0000x reference)
import jax
import jax.numpy as jnp
from jax import lax
from jax.experimental import pallas as pl
from jax.experimental.pallas import tpu as pltpu
from jax.experimental.pallas import tpu_sc as plsc

_F = 26
_V = 100000
_E = 64
_B = 4096

_NC = 2
_NS = 16
_NW = _NC * _NS

_TOTAL = _B * _F
_PER_W = _TOTAL // _NW
_LANES = 16
_NSLICE = _PER_W // _LANES
_CHUNK = 128
_NCHUNK = _PER_W // _CHUNK


def _body(x_hbm, tab_hbm, out_hbm, idx_v, rows_v, gsem):
    wid = lax.axis_index("s") * _NC + lax.axis_index("c")
    base = wid * _PER_W

    pltpu.sync_copy(x_hbm.at[pl.ds(base, _PER_W)], idx_v)

    def adjust(i, carry):
        p0 = i * _LANES
        vals = idx_v[pl.ds(p0, _LANES)]
        pos = lax.iota(jnp.int32, _LANES) + p0
        field = lax.rem(pos, _F)
        idx_v[pl.ds(p0, _LANES)] = vals + field * _V
        return carry

    lax.fori_loop(0, _NSLICE, adjust, 0)

    def chunk(c, carry):
        cb = c * _CHUNK
        pltpu.async_copy(
            tab_hbm.at[idx_v.at[pl.ds(cb, _CHUNK)]], rows_v, gsem
        ).wait()
        pltpu.sync_copy(rows_v, out_hbm.at[pl.ds(base + cb, _CHUNK)])
        return carry

    lax.fori_loop(0, _NCHUNK, chunk, 0)


@jax.jit
def kernel(X, tables):
    xflat = X.reshape(_TOTAL)
    tab = tables.reshape(_F * _V, _E)
    run = pl.kernel(
        _body,
        out_type=jax.ShapeDtypeStruct((_TOTAL, _E), jnp.float32),
        mesh=plsc.VectorSubcoreMesh(core_axis_name="c", subcore_axis_name="s"),
        compiler_params=pltpu.CompilerParams(use_tc_tiling_on_sc=False),
        scratch_types=[
            pltpu.VMEM((_PER_W,), jnp.int32),
            pltpu.VMEM((_CHUNK, _E), jnp.float32),
            pltpu.SemaphoreType.DMA,
        ],
    )
    out = run(xflat, tab)
    return out.reshape(_B, 1, _F * _E)


# speedup vs baseline: 1.0105x; 1.0105x over previous
import jax
import jax.numpy as jnp
from jax import lax
from jax.experimental import pallas as pl
from jax.experimental.pallas import tpu as pltpu
from jax.experimental.pallas import tpu_sc as plsc

_F = 26
_V = 100000
_E = 64
_B = 4096

_NC = 2
_NS = 16
_NW = _NC * _NS

_TOTAL = _B * _F
_PER_W = _TOTAL // _NW
_LANES = 16
_CHUNK = 416
_NCHUNK = _PER_W // _CHUNK
_NBUF = 4
_SL_PER_CHUNK = _CHUNK // _LANES


def _adjust(idx_v, cb):
    def f(i, carry):
        p0 = cb + i * _LANES
        vals = idx_v[pl.ds(p0, _LANES)]
        fld = lax.rem(lax.iota(jnp.int32, _LANES) + p0, _F)
        idx_v[pl.ds(p0, _LANES)] = vals + fld * _V
        return carry

    lax.fori_loop(0, _SL_PER_CHUNK, f, 0)


def _body(x_hbm, tab_hbm, out_hbm, idx_v, r0, r1, r2, r3, s0, s1, s2, s3):
    rows = (r0, r1, r2, r3)
    sems = (s0, s1, s2, s3)
    wid = lax.axis_index("s") * _NC + lax.axis_index("c")
    base = wid * _PER_W

    pltpu.sync_copy(x_hbm.at[pl.ds(base, _PER_W)], idx_v)

    g = {}
    for c in range(_NBUF):
        _adjust(idx_v, c * _CHUNK)
        g[c] = pltpu.async_copy(
            tab_hbm.at[idx_v.at[pl.ds(c * _CHUNK, _CHUNK)]], rows[c], sems[c]
        )
    for c in range(_NBUF, _NCHUNK):
        _adjust(idx_v, c * _CHUNK)

    for c in range(_NCHUNK):
        b = c % _NBUF
        g[c].wait()
        pltpu.sync_copy(rows[b], out_hbm.at[pl.ds(base + c * _CHUNK, _CHUNK)])
        nc = c + _NBUF
        if nc < _NCHUNK:
            g[nc] = pltpu.async_copy(
                tab_hbm.at[idx_v.at[pl.ds(nc * _CHUNK, _CHUNK)]],
                rows[b],
                sems[b],
            )


@jax.jit
def kernel(X, tables):
    xflat = X.reshape(_TOTAL)
    tab = tables.reshape(_F * _V, _E)
    run = pl.kernel(
        _body,
        out_type=jax.ShapeDtypeStruct((_TOTAL, _E), jnp.float32),
        mesh=plsc.VectorSubcoreMesh(core_axis_name="c", subcore_axis_name="s"),
        compiler_params=pltpu.CompilerParams(use_tc_tiling_on_sc=False),
        scratch_types=[
            pltpu.VMEM((_PER_W,), jnp.int32),
            pltpu.VMEM((_CHUNK, _E), jnp.float32),
            pltpu.VMEM((_CHUNK, _E), jnp.float32),
            pltpu.VMEM((_CHUNK, _E), jnp.float32),
            pltpu.VMEM((_CHUNK, _E), jnp.float32),
            pltpu.SemaphoreType.DMA,
            pltpu.SemaphoreType.DMA,
            pltpu.SemaphoreType.DMA,
            pltpu.SemaphoreType.DMA,
        ],
    )
    out = run(xflat, tab)
    return out.reshape(_B, 1, _F * _E)


# speedup vs baseline: 2.0593x; 2.0379x over previous
import jax
import jax.numpy as jnp
from jax import lax
from jax.experimental import pallas as pl
from jax.experimental.pallas import tpu as pltpu
from jax.experimental.pallas import tpu_sc as plsc

_F = 26
_V = 100000
_E = 64
_B = 4096
_NC = 2
_NS = 16
_NW = _NC * _NS
_TOTAL = _B * _F
_PER_W = _TOTAL // _NW
_G = 16
_NGRP = _PER_W // _G
_FLUSH = 128


def _fire(tab_hbm, vvec, p0, tbuf, sem):
    copies = []
    for j in range(_G):
        v = vvec[j]
        f = lax.rem(p0 + j, _F)
        row = f * _V + v
        r8 = pl.multiple_of((row >> 3) << 3, 8)
        copies.append(
            (
                pltpu.async_copy(
                    tab_hbm.at[pl.ds(r8, 8), :], tbuf.at[j], sem
                ),
                lax.rem(v, 8),
            )
        )
    return copies


def _drain(copies, tbuf, obuf, o0):
    for j in range(_G):
        copies[j][0].wait()
    for j in range(_G):
        r = copies[j][1]
        for k in range(_E // 16):
            obuf[pl.ds(o0 + j * _E + k * 16, 16)] = tbuf[j, r, pl.ds(k * 16, 16)]


def _body(x_hbm, tab_hbm, out_hbm, idx_v, tb0, tb1, ob0, ob1, s0, s1, w0, w1):
    wid = lax.axis_index("s") * _NC + lax.axis_index("c")
    base = wid * _PER_W
    pltpu.sync_copy(x_hbm.at[pl.ds(base, _PER_W)], idx_v)
    tbs = (tb0, tb1)
    obs = (ob0, ob1)
    sems = (s0, s1)
    wsems = (w0, w1)

    def one_block(p_blk, ob):
        obuf = obs[ob]
        prev = None
        for w in range(_FLUSH // _G):
            p0 = p_blk + w * _G
            vvec = idx_v[pl.ds(p0, _G)]
            cur = _fire(tab_hbm, vvec, base + p0, tbs[w % 2], sems[w % 2])
            if prev is not None:
                _drain(prev[0], tbs[(w + 1) % 2], obuf, prev[1])
            prev = (cur, w * _G * _E)
        _drain(prev[0], tbs[(_FLUSH // _G - 1) % 2], obuf, prev[1])
        pltpu.async_copy(
            obuf, out_hbm.at[pl.ds((base + p_blk) * _E, _FLUSH * _E)], wsems[ob]
        ).wait()

    def block_pair(bp, carry):
        one_block(bp * 2 * _FLUSH, 0)
        one_block((bp * 2 + 1) * _FLUSH, 1)
        return carry

    lax.fori_loop(0, _PER_W // _FLUSH // 2, block_pair, 0)


@jax.jit
def kernel(X, tables):
    xflat = X.reshape(_TOTAL)
    run = pl.kernel(
        _body,
        out_type=jax.ShapeDtypeStruct((_TOTAL * _E,), jnp.float32),
        mesh=plsc.VectorSubcoreMesh(core_axis_name="c", subcore_axis_name="s"),
        compiler_params=pltpu.CompilerParams(use_tc_tiling_on_sc=True),
        scratch_types=[
            pltpu.VMEM((_PER_W,), jnp.int32),
            pltpu.VMEM((_G, 8, _E), jnp.float32),
            pltpu.VMEM((_G, 8, _E), jnp.float32),
            pltpu.VMEM((_FLUSH * _E,), jnp.float32),
            pltpu.VMEM((_FLUSH * _E,), jnp.float32),
            pltpu.SemaphoreType.DMA,
            pltpu.SemaphoreType.DMA,
            pltpu.SemaphoreType.DMA,
            pltpu.SemaphoreType.DMA,
        ],
    )
    out = run(xflat, tables.reshape(_F * _V, _E))
    return out.reshape(_B, 1, _F * _E)


# speedup vs baseline: 2.0718x; 1.0061x over previous
import jax
import jax.numpy as jnp
from jax import lax
from jax.experimental import pallas as pl
from jax.experimental.pallas import tpu as pltpu
from jax.experimental.pallas import tpu_sc as plsc

_F = 26
_V = 100000
_E = 64
_B = 4096
_NC = 2
_NS = 16
_NW = _NC * _NS
_TOTAL = _B * _F
_PER_W = _TOTAL // _NW
_G = 16
_WPB = 8
_RING = 4
_FLUSH = _G * _WPB
_NBLK = _PER_W // _FLUSH


def _fire_wave(tab_hbm, row_v, p0, tbuf, sem):
    rvec = row_v[pl.ds(p0, _G)]
    for j in range(_G):
        row = rvec[j]
        r8 = pl.multiple_of((row >> 3) << 3, 8)
        pltpu.async_copy(
            tab_hbm.at[pl.ds(r8, 8), :], tbuf.at[pl.ds(j * 8, 8), :], sem
        )


def _drain_wave(tab_hbm, row_v, p0, tbuf, sem, obuf, o0):
    pltpu.make_async_copy(tab_hbm.at[pl.ds(0, _G * 8), :], tbuf, sem).wait()
    rvec = row_v[pl.ds(p0, _G)]
    for j in range(_G):
        r = j * 8 + lax.rem(rvec[j], 8)
        for k in range(_E // 16):
            obuf[pl.ds(o0 + j * _E + k * 16, 16)] = tbuf[r, pl.ds(k * 16, 16)]


def _body(x_hbm, tab_hbm, out_hbm, idx_v, *rest):
    tbufs = rest[:_RING]
    ob0, ob1 = rest[_RING], rest[_RING + 1]
    sems = rest[_RING + 2 : 2 * _RING + 2]
    wsems = rest[2 * _RING + 2 : 2 * _RING + 4]
    obs = (ob0, ob1)

    wid = lax.axis_index("s") * _NC + lax.axis_index("c")
    base = wid * _PER_W
    pltpu.sync_copy(x_hbm.at[pl.ds(base, _PER_W)], idx_v.at[pl.ds(0, _PER_W)])

    def adjust(i, carry):
        p0 = i * _G
        vals = idx_v[pl.ds(p0, _G)]
        fld = lax.rem(lax.iota(jnp.int32, _G) + p0, _F)
        idx_v[pl.ds(p0, _G)] = vals + fld * _V
        return carry

    lax.fori_loop(0, _PER_W // _G, adjust, 0)

    for i in range(_FLUSH // _G):
        idx_v[pl.ds(_PER_W + i * _G, _G)] = jnp.zeros((_G,), jnp.int32)

    def do_block(p_blk, ob, guard):
        obuf = obs[ob]

        @pl.when(guard)
        def _wait_prev_flush():
            pltpu.make_async_copy(
                out_hbm.at[pl.ds(0, _FLUSH * _E)], obuf, wsems[ob]
            ).wait()

        for w in range(_WPB):
            slot = w % _RING
            _drain_wave(tab_hbm, idx_v, p_blk + w * _G, tbufs[slot], sems[slot], obuf, w * _G * _E)
            _fire_wave(tab_hbm, idx_v, p_blk + (w + _RING) * _G, tbufs[slot], sems[slot])
        pltpu.async_copy(
            obuf, out_hbm.at[pl.ds((base + p_blk) * _E, _FLUSH * _E)], wsems[ob]
        )

    for w in range(_RING):
        _fire_wave(tab_hbm, idx_v, w * _G, tbufs[w], sems[w])

    def pair(bp, carry):
        p = bp * 2 * _FLUSH
        do_block(p, 0, bp > 0)
        do_block(p + _FLUSH, 1, bp > 0)
        return carry

    lax.fori_loop(0, _NBLK // 2, pair, 0)

    for w in range(_RING):
        pltpu.make_async_copy(
            tab_hbm.at[pl.ds(0, _G * 8), :], tbufs[w], sems[w]
        ).wait()
    for ob in range(2):
        pltpu.make_async_copy(
            out_hbm.at[pl.ds(0, _FLUSH * _E)], obs[ob], wsems[ob]
        ).wait()


@jax.jit
def kernel(X, tables):
    xflat = X.reshape(_TOTAL)
    run = pl.kernel(
        _body,
        out_type=jax.ShapeDtypeStruct((_TOTAL * _E,), jnp.float32),
        mesh=plsc.VectorSubcoreMesh(core_axis_name="c", subcore_axis_name="s"),
        compiler_params=pltpu.CompilerParams(use_tc_tiling_on_sc=True),
        scratch_types=[pltpu.VMEM((_PER_W + _FLUSH,), jnp.int32)]
        + [pltpu.VMEM((_G * 8, _E), jnp.float32) for _ in range(_RING)]
        + [pltpu.VMEM((_FLUSH * _E,), jnp.float32) for _ in range(2)]
        + [pltpu.SemaphoreType.DMA for _ in range(_RING + 2)],
    )
    out = run(xflat, tables.reshape(_F * _V, _E))
    return out.reshape(_B, 1, _F * _E)
